# baseline (speedup 1.0000x reference)
import jax
import jax.numpy as jnp
from jax import lax
from jax.experimental import pallas as pl
from jax.experimental.pallas import tpu as pltpu
from jax.experimental.pallas import tpu_sc as plsc

_N = 10000
_E = 320000
_D = 128
_CHUNK = 2500
_ROWS = 5120
_G = 80
_SB = 4000
_NSUB = 16
_EPS = _E // _NSUB
_NSB = _EPS // _SB
_TPB = _SB // _G
_ZR = 16
_RPS = _ROWS // _NSUB
_OUT_ROWS = 2 * _CHUNK


def _sc_ab_body(dst_h, inv_h, reli_h, srci_h, rel_emb_h, ent_emb_h,
                arel_h, aent_h,
                st_dst, st_inv, st_rel, st_src,
                b_acc, b_rel, b_src,
                g_rel, g_ent, zbuf,
                arel_sh, aent_sh, sem0, sem1):
  c = lax.axis_index("c")
  s = lax.axis_index("s")
  i16 = lax.iota(jnp.int32, 16)
  zeros16 = jnp.zeros((16,), jnp.float32)
  dummy_acc = 2 * _CHUNK + i16 * 7

  for rr_ in range(_ZR):
    for cc in range(_D // 16):
      zbuf[rr_, pl.ds(cc * 16, 16)] = zeros16

  for p in range(2):
    chunk = c * 2 + p
    base = chunk * _CHUNK

    z0 = s * _RPS
    for k in range(_RPS // _ZR):
      rr = pl.ds(z0 + k * _ZR, _ZR)
      pltpu.sync_copy(zbuf, arel_sh.at[rr])
      pltpu.sync_copy(zbuf, aent_sh.at[rr])
    plsc.subcore_barrier()

    for sb in range(_NSB):
      eb = s * _EPS + sb * _SB
      pltpu.sync_copy(dst_h.at[pl.ds(eb, _SB)], st_dst)
      pltpu.sync_copy(inv_h.at[pl.ds(eb, _SB)], st_inv)
      pltpu.sync_copy(reli_h.at[pl.ds(eb, _SB)], st_rel)
      pltpu.sync_copy(srci_h.at[pl.ds(eb, _SB)], st_src)

      def _tile(t, carry):
        for v in range(_G // 16):
          off = t * _G + v * 16
          dv = st_dst[pl.ds(off, 16)]
          iv = st_inv[pl.ds(off, 16)]
          rv = st_rel[pl.ds(off, 16)]
          sv = st_src[pl.ds(off, 16)]
          m = (dv >= base) & (dv < base + _CHUNK)
          b_acc[pl.ds(v * 16, 16)] = jnp.where(m, (dv - base) * 2 + iv,
                                               dummy_acc)
          b_rel[pl.ds(v * 16, 16)] = jnp.where(m, rv, i16)
          b_src[pl.ds(v * 16, 16)] = jnp.where(m, sv, i16)
        cp0 = pltpu.async_copy(rel_emb_h.at[b_rel], g_rel, sem0)
        cp1 = pltpu.async_copy(ent_emb_h.at[b_src], g_ent, sem1)
        cp0.wait()
        cp1.wait()
        pltpu.sync_copy(g_rel, arel_sh.at[b_acc], add=True)
        pltpu.sync_copy(g_ent, aent_sh.at[b_acc], add=True)
        return carry

      lax.fori_loop(0, _TPB, _tile, jnp.int32(0))

    plsc.subcore_barrier()
    rr = pl.ds(s * _RPS, _RPS)
    pltpu.sync_copy(arel_sh.at[rr], arel_h.at[chunk, rr])
    pltpu.sync_copy(aent_sh.at[rr], aent_h.at[chunk, rr])
    plsc.subcore_barrier()


_sc_agg_ab = pl.kernel(
    _sc_ab_body,
    out_type=(
        jax.ShapeDtypeStruct((4, _ROWS, _D), jnp.float32),
        jax.ShapeDtypeStruct((4, _ROWS, _D), jnp.float32),
    ),
    mesh=plsc.VectorSubcoreMesh(core_axis_name="c", subcore_axis_name="s"),
    scratch_types=[
        pltpu.VMEM((_SB,), jnp.int32),
        pltpu.VMEM((_SB,), jnp.int32),
        pltpu.VMEM((_SB,), jnp.int32),
        pltpu.VMEM((_SB,), jnp.int32),
        pltpu.VMEM((_G,), jnp.int32),
        pltpu.VMEM((_G,), jnp.int32),
        pltpu.VMEM((_G,), jnp.int32),
        pltpu.VMEM((_G, _D), jnp.float32),
        pltpu.VMEM((_G, _D), jnp.float32),
        pltpu.VMEM((_ZR, _D), jnp.float32),
        pltpu.VMEM_SHARED((_ROWS, _D), jnp.float32),
        pltpu.VMEM_SHARED((_ROWS, _D), jnp.float32),
        pltpu.SemaphoreType.DMA,
        pltpu.SemaphoreType.DMA,
    ],
)


def _sc_c_body(dst_h, inv_h, timi_h, tim_emb_h, atim_h, cnt_h,
               st_dst, st_inv, st_tim,
               b_acc, b_tim,
               g_tim, ones_b, zbuf,
               atim_sh, cnt_sh, sem0):
  c = lax.axis_index("c")
  s = lax.axis_index("s")
  i16 = lax.iota(jnp.int32, 16)
  ones16 = jnp.ones((16,), jnp.float32)
  zeros16 = jnp.zeros((16,), jnp.float32)
  dummy_acc = 2 * _CHUNK + i16 * 7

  for rr_ in range(_ZR):
    for cc in range(_D // 16):
      zbuf[rr_, pl.ds(cc * 16, 16)] = zeros16
  for rr_ in range(_G):
    for cc in range(_D // 16):
      ones_b[rr_, pl.ds(cc * 16, 16)] = ones16

  for p in range(2):
    chunk = c * 2 + p
    base = chunk * _CHUNK

    z0 = s * _RPS
    for k in range(_RPS // _ZR):
      rr = pl.ds(z0 + k * _ZR, _ZR)
      pltpu.sync_copy(zbuf, atim_sh.at[rr])
      pltpu.sync_copy(zbuf, cnt_sh.at[rr])
    plsc.subcore_barrier()

    for sb in range(_NSB):
      eb = s * _EPS + sb * _SB
      pltpu.sync_copy(dst_h.at[pl.ds(eb, _SB)], st_dst)
      pltpu.sync_copy(inv_h.at[pl.ds(eb, _SB)], st_inv)
      pltpu.sync_copy(timi_h.at[pl.ds(eb, _SB)], st_tim)

      def _tile(t, carry):
        for v in range(_G // 16):
          off = t * _G + v * 16
          dv = st_dst[pl.ds(off, 16)]
          iv = st_inv[pl.ds(off, 16)]
          tv = st_tim[pl.ds(off, 16)]
          m = (dv >= base) & (dv < base + _CHUNK)
          b_acc[pl.ds(v * 16, 16)] = jnp.where(m, (dv - base) * 2 + iv,
                                               dummy_acc)
          b_tim[pl.ds(v * 16, 16)] = jnp.where(m, tv, i16)
        cp0 = pltpu.async_copy(tim_emb_h.at[b_tim], g_tim, sem0)
        cp0.wait()
        pltpu.sync_copy(g_tim, atim_sh.at[b_acc], add=True)
        pltpu.sync_copy(ones_b, cnt_sh.at[b_acc], add=True)
        return carry

      lax.fori_loop(0, _TPB, _tile, jnp.int32(0))

    plsc.subcore_barrier()
    rr = pl.ds(s * _RPS, _RPS)
    pltpu.sync_copy(atim_sh.at[rr], atim_h.at[chunk, rr])
    pltpu.sync_copy(cnt_sh.at[rr], cnt_h.at[chunk, rr])
    plsc.subcore_barrier()


_sc_agg_c = pl.kernel(
    _sc_c_body,
    out_type=(
        jax.ShapeDtypeStruct((4, _ROWS, _D), jnp.float32),
        jax.ShapeDtypeStruct((4, _ROWS, _D), jnp.float32),
    ),
    mesh=plsc.VectorSubcoreMesh(core_axis_name="c", subcore_axis_name="s"),
    scratch_types=[
        pltpu.VMEM((_SB,), jnp.int32),
        pltpu.VMEM((_SB,), jnp.int32),
        pltpu.VMEM((_SB,), jnp.int32),
        pltpu.VMEM((_G,), jnp.int32),
        pltpu.VMEM((_G,), jnp.int32),
        pltpu.VMEM((_G, _D), jnp.float32),
        pltpu.VMEM((_G, _D), jnp.float32),
        pltpu.VMEM((_ZR, _D), jnp.float32),
        pltpu.VMEM_SHARED((_ROWS, _D), jnp.float32),
        pltpu.VMEM_SHARED((_ROWS, _D), jnp.float32),
        pltpu.SemaphoreType.DMA,
    ],
)


_BLK = 1000


def _combine_body(arel_ref, aent_ref, atim_ref, ent_ref, aux_ref, wr_ref,
                  ws_ref, wt_ref, wself_ref, bio_ref, bs_ref, o_ref):
  hi = lax.Precision.HIGHEST
  acc = jnp.dot(arel_ref[...], wr_ref[...], precision=hi,
                preferred_element_type=jnp.float32)
  acc = acc + jnp.dot(aent_ref[...], ws_ref[...], precision=hi,
                      preferred_element_type=jnp.float32)
  acc = acc + jnp.dot(atim_ref[...], wt_ref[...], precision=hi,
                      preferred_element_type=jnp.float32)
  aux = aux_ref[...]
  bio = bio_ref[...]
  acc = acc + aux[:, 0:1] * bio[0:1, :] + aux[:, 1:2] * bio[1:2, :]
  acc = acc * aux[:, 2:3]
  o_ref[...] = acc + jnp.dot(ent_ref[...], wself_ref[...], precision=hi,
                             preferred_element_type=jnp.float32) + bs_ref[...]


def _tc_combine(arel2, aent2, atim2, ent_emb, aux, wr, ws, wt, wself, bio,
                bs):
  grid = _N // _BLK
  return pl.pallas_call(
      _combine_body,
      grid=(grid,),
      in_specs=[
          pl.BlockSpec((_BLK, 2 * _D), lambda i: (i, 0)),
          pl.BlockSpec((_BLK, 2 * _D), lambda i: (i, 0)),
          pl.BlockSpec((_BLK, 2 * _D), lambda i: (i, 0)),
          pl.BlockSpec((_BLK, _D), lambda i: (i, 0)),
          pl.BlockSpec((_BLK, 8), lambda i: (i, 0)),
          pl.BlockSpec((2 * _D, _D), lambda i: (0, 0)),
          pl.BlockSpec((2 * _D, _D), lambda i: (0, 0)),
          pl.BlockSpec((2 * _D, _D), lambda i: (0, 0)),
          pl.BlockSpec((_D, _D), lambda i: (0, 0)),
          pl.BlockSpec((2, _D), lambda i: (0, 0)),
          pl.BlockSpec((1, _D), lambda i: (0, 0)),
      ],
      out_specs=pl.BlockSpec((_BLK, _D), lambda i: (i, 0)),
      out_shape=jax.ShapeDtypeStruct((_N, _D), jnp.float32),
  )(arel2, aent2, atim2, ent_emb, aux, wr, ws, wt, wself, bio, bs)


def _lin_body(x_ref, w_ref, b_ref, o_ref):
  o_ref[...] = jnp.dot(x_ref[...], w_ref[...], precision=lax.Precision.HIGHEST,
                       preferred_element_type=jnp.float32) + b_ref[...]


def _tc_lin(x, wt, b):
  return pl.pallas_call(
      _lin_body,
      out_shape=jax.ShapeDtypeStruct((x.shape[0], _D), jnp.float32),
  )(x, wt, b)


def kernel(ent_emb, rel_emb, time_emb, edge_index, b_rel, e_time, inv, Wi_w,
           Wi_b, Wo_w, Wo_b, Ws_w, Ws_b, Wr_w, Wr_b, Wt_w, Wt_b):
  src = edge_index[0]
  dst = edge_index[1]
  arel, aent = _sc_agg_ab(dst, inv, b_rel, src, rel_emb, ent_emb)
  atim, cnt = _sc_agg_c(dst, inv, e_time, time_emb)
  arel2 = arel[:, :_OUT_ROWS].reshape(_N, 2 * _D)
  aent2 = aent[:, :_OUT_ROWS].reshape(_N, 2 * _D)
  atim2 = atim[:, :_OUT_ROWS].reshape(_N, 2 * _D)
  cnt2 = cnt[:, :_OUT_ROWS, 0].reshape(_N, 2)
  ci = cnt2[:, 0:1]
  co = cnt2[:, 1:2]
  invd = 1.0 / jnp.maximum(ci + co, 1.0)
  aux = jnp.concatenate([ci, co, invd, jnp.zeros((_N, 5), jnp.float32)],
                        axis=1)
  wr = jnp.concatenate([Wi_w[:, 0:_D].T, Wo_w[:, 0:_D].T], axis=0)
  ws = jnp.concatenate([Wi_w[:, _D:2 * _D].T, Wo_w[:, _D:2 * _D].T], axis=0)
  wt = jnp.concatenate([Wi_w[:, 2 * _D:].T, Wo_w[:, 2 * _D:].T], axis=0)
  bio = jnp.stack([Wi_b, Wo_b])
  ent_new = _tc_combine(arel2, aent2, atim2, ent_emb, aux, wr, ws, wt,
                        Ws_w.T, bio, Ws_b[None])
  rel_new = _tc_lin(rel_emb, Wr_w.T, Wr_b[None])
  time_new = _tc_lin(time_emb, Wt_w.T, Wt_b[None])
  return ent_new, rel_new, time_new


# speedup vs baseline: 1.0157x; 1.0157x over previous
import jax
import jax.numpy as jnp
from jax import lax
from jax.experimental import pallas as pl
from jax.experimental.pallas import tpu as pltpu
from jax.experimental.pallas import tpu_sc as plsc

_N = 10000
_E = 320000
_D = 128
_CHUNK = 2500
_ROWS = 5120
_G = 80
_SB = 4000
_NSUB = 16
_EPS = _E // _NSUB
_NSB = _EPS // _SB
_TPB = _SB // _G
_ZR = 16
_RPS = _ROWS // _NSUB
_OUT_ROWS = 2 * _CHUNK


_TPA = 125


def _sc_ab_body(dst_h, inv_h, reli_h, srci_h, comb_h, acc_out_h,
                st_dst, st_inv, st_rel, st_src,
                b_acc0, b_idx0, b_acc1, b_idx1,
                g0, g1, zbuf,
                acc_sh, sg0, sg1):
  c = lax.axis_index("c")
  s = lax.axis_index("s")
  i16 = lax.iota(jnp.int32, 16)
  zeros16 = jnp.zeros((16,), jnp.float32)
  dummy_acc = 2 * _CHUNK + i16 * 7

  for rr_ in range(_ZR):
    for cc in range(_D // 16):
      zbuf[rr_, pl.ds(cc * 16, 16)] = zeros16

  for p in range(2):
    chunk = c * 2 + p
    base = chunk * _CHUNK

    z0 = s * (2 * _RPS)
    for k in range(2 * _RPS // _ZR):
      pltpu.sync_copy(zbuf, acc_sh.at[pl.ds(z0 + k * _ZR, _ZR)])
    plsc.subcore_barrier()

    for sb in range(_NSB):
      eb = s * _EPS + sb * _SB
      pltpu.sync_copy(dst_h.at[pl.ds(eb, _SB)], st_dst)
      pltpu.sync_copy(inv_h.at[pl.ds(eb, _SB)], st_inv)
      pltpu.sync_copy(reli_h.at[pl.ds(eb, _SB)], st_rel)
      pltpu.sync_copy(srci_h.at[pl.ds(eb, _SB)], st_src)

      def _build(t, ba, bi):
        for v in range(2):
          off = t * 32 + v * 16
          dv = st_dst[pl.ds(off, 16)]
          iv = st_inv[pl.ds(off, 16)]
          rv = st_rel[pl.ds(off, 16)]
          sv = st_src[pl.ds(off, 16)]
          m = (dv >= base) & (dv < base + _CHUNK)
          a = jnp.where(m, (dv - base) * 2 + iv, dummy_acc)
          bi[pl.ds(v * 16, 16)] = jnp.where(m, rv, i16)
          bi[pl.ds(32 + v * 16, 16)] = jnp.where(m, 500 + sv, 500 + i16)
          ba[pl.ds(v * 16, 16)] = a
          ba[pl.ds(32 + v * 16, 16)] = _ROWS + a

      def _tile2(j, carry):
        t0 = j * 2
        _build(t0, b_acc0, b_idx0)
        cg0 = pltpu.async_copy(comb_h.at[b_idx0], g0, sg0)
        _build(t0 + 1, b_acc1, b_idx1)
        cg1 = pltpu.async_copy(comb_h.at[b_idx1], g1, sg1)
        cg0.wait()
        pltpu.sync_copy(g0, acc_sh.at[b_acc0], add=True)
        cg1.wait()
        pltpu.sync_copy(g1, acc_sh.at[b_acc1], add=True)
        return carry

      lax.fori_loop(0, _TPA // 2, _tile2, jnp.int32(0))
      _build(_TPA - 1, b_acc0, b_idx0)
      cg0 = pltpu.async_copy(comb_h.at[b_idx0], g0, sg0)
      cg0.wait()
      pltpu.sync_copy(g0, acc_sh.at[b_acc0], add=True)

    plsc.subcore_barrier()
    rr = pl.ds(s * (2 * _RPS), 2 * _RPS)
    pltpu.sync_copy(acc_sh.at[rr], acc_out_h.at[chunk, rr])
    plsc.subcore_barrier()


_sc_agg_ab = pl.kernel(
    _sc_ab_body,
    out_type=jax.ShapeDtypeStruct((4, 2 * _ROWS, _D), jnp.float32),
    mesh=plsc.VectorSubcoreMesh(core_axis_name="c", subcore_axis_name="s"),
    scratch_types=[
        pltpu.VMEM((_SB,), jnp.int32),
        pltpu.VMEM((_SB,), jnp.int32),
        pltpu.VMEM((_SB,), jnp.int32),
        pltpu.VMEM((_SB,), jnp.int32),
        pltpu.VMEM((64,), jnp.int32),
        pltpu.VMEM((64,), jnp.int32),
        pltpu.VMEM((64,), jnp.int32),
        pltpu.VMEM((64,), jnp.int32),
        pltpu.VMEM((64, _D), jnp.float32),
        pltpu.VMEM((64, _D), jnp.float32),
        pltpu.VMEM((_ZR, _D), jnp.float32),
        pltpu.VMEM_SHARED((2 * _ROWS, _D), jnp.float32),
        pltpu.SemaphoreType.DMA,
        pltpu.SemaphoreType.DMA,
    ],
)


def _sc_c_body(dst_h, inv_h, timi_h, tim_emb_h, atim_h, cnt_h,
               st_dst, st_inv, st_tim,
               b_acc0, b_tim0, b_acc1, b_tim1,
               g_tim0, g_tim1, ones_b, zbuf,
               atim_sh, cnt_sh,
               sg0, sg1, ss0, ss1):
  c = lax.axis_index("c")
  s = lax.axis_index("s")
  i16 = lax.iota(jnp.int32, 16)
  ones16 = jnp.ones((16,), jnp.float32)
  zeros16 = jnp.zeros((16,), jnp.float32)
  dummy_acc = 2 * _CHUNK + i16 * 7

  for rr_ in range(_ZR):
    for cc in range(_D // 16):
      zbuf[rr_, pl.ds(cc * 16, 16)] = zeros16
  for rr_ in range(_G):
    for cc in range(_D // 16):
      ones_b[rr_, pl.ds(cc * 16, 16)] = ones16

  for p in range(2):
    chunk = c * 2 + p
    base = chunk * _CHUNK

    z0 = s * _RPS
    for k in range(_RPS // _ZR):
      rr = pl.ds(z0 + k * _ZR, _ZR)
      pltpu.sync_copy(zbuf, atim_sh.at[rr])
      pltpu.sync_copy(zbuf, cnt_sh.at[rr])
    plsc.subcore_barrier()

    for sb in range(_NSB):
      eb = s * _EPS + sb * _SB
      pltpu.sync_copy(dst_h.at[pl.ds(eb, _SB)], st_dst)
      pltpu.sync_copy(inv_h.at[pl.ds(eb, _SB)], st_inv)
      pltpu.sync_copy(timi_h.at[pl.ds(eb, _SB)], st_tim)

      def _build(t, ba, bt):
        for v in range(_G // 16):
          off = t * _G + v * 16
          dv = st_dst[pl.ds(off, 16)]
          iv = st_inv[pl.ds(off, 16)]
          tv = st_tim[pl.ds(off, 16)]
          m = (dv >= base) & (dv < base + _CHUNK)
          ba[pl.ds(v * 16, 16)] = jnp.where(m, (dv - base) * 2 + iv,
                                            dummy_acc)
          bt[pl.ds(v * 16, 16)] = jnp.where(m, tv, i16)

      def _tile2(j, carry):
        t0 = j * 2
        _build(t0, b_acc0, b_tim0)
        cg0 = pltpu.async_copy(tim_emb_h.at[b_tim0], g_tim0, sg0)
        _build(t0 + 1, b_acc1, b_tim1)
        cg1 = pltpu.async_copy(tim_emb_h.at[b_tim1], g_tim1, sg1)
        cg0.wait()
        cs0 = pltpu.async_copy(g_tim0, atim_sh.at[b_acc0], ss0, add=True)
        cs1 = pltpu.async_copy(ones_b, cnt_sh.at[b_acc0], ss0, add=True)
        cg1.wait()
        cs2 = pltpu.async_copy(g_tim1, atim_sh.at[b_acc1], ss1, add=True)
        cs3 = pltpu.async_copy(ones_b, cnt_sh.at[b_acc1], ss1, add=True)
        cs0.wait()
        cs1.wait()
        cs2.wait()
        cs3.wait()
        return carry

      lax.fori_loop(0, _TPB // 2, _tile2, jnp.int32(0))

    plsc.subcore_barrier()
    rr = pl.ds(s * _RPS, _RPS)
    pltpu.sync_copy(atim_sh.at[rr], atim_h.at[chunk, rr])
    pltpu.sync_copy(cnt_sh.at[rr], cnt_h.at[chunk, rr])
    plsc.subcore_barrier()


_sc_agg_c = pl.kernel(
    _sc_c_body,
    out_type=(
        jax.ShapeDtypeStruct((4, _ROWS, _D), jnp.float32),
        jax.ShapeDtypeStruct((4, _ROWS, _D), jnp.float32),
    ),
    mesh=plsc.VectorSubcoreMesh(core_axis_name="c", subcore_axis_name="s"),
    scratch_types=[
        pltpu.VMEM((_SB,), jnp.int32),
        pltpu.VMEM((_SB,), jnp.int32),
        pltpu.VMEM((_SB,), jnp.int32),
        pltpu.VMEM((_G,), jnp.int32),
        pltpu.VMEM((_G,), jnp.int32),
        pltpu.VMEM((_G,), jnp.int32),
        pltpu.VMEM((_G,), jnp.int32),
        pltpu.VMEM((_G, _D), jnp.float32),
        pltpu.VMEM((_G, _D), jnp.float32),
        pltpu.VMEM((_G, _D), jnp.float32),
        pltpu.VMEM((_ZR, _D), jnp.float32),
        pltpu.VMEM_SHARED((_ROWS, _D), jnp.float32),
        pltpu.VMEM_SHARED((_ROWS, _D), jnp.float32),
        pltpu.SemaphoreType.DMA,
        pltpu.SemaphoreType.DMA,
        pltpu.SemaphoreType.DMA,
        pltpu.SemaphoreType.DMA,
    ],
)


_BLK = 1000


def _combine_body(arel_ref, aent_ref, atim_ref, ent_ref, aux_ref, wr_ref,
                  ws_ref, wt_ref, wself_ref, bio_ref, bs_ref, o_ref):
  hi = lax.Precision.HIGHEST
  acc = jnp.dot(arel_ref[...], wr_ref[...], precision=hi,
                preferred_element_type=jnp.float32)
  acc = acc + jnp.dot(aent_ref[...], ws_ref[...], precision=hi,
                      preferred_element_type=jnp.float32)
  acc = acc + jnp.dot(atim_ref[...], wt_ref[...], precision=hi,
                      preferred_element_type=jnp.float32)
  aux = aux_ref[...]
  bio = bio_ref[...]
  acc = acc + aux[:, 0:1] * bio[0:1, :] + aux[:, 1:2] * bio[1:2, :]
  acc = acc * aux[:, 2:3]
  o_ref[...] = acc + jnp.dot(ent_ref[...], wself_ref[...], precision=hi,
                             preferred_element_type=jnp.float32) + bs_ref[...]


def _tc_combine(arel2, aent2, atim2, ent_emb, aux, wr, ws, wt, wself, bio,
                bs):
  grid = _N // _BLK
  return pl.pallas_call(
      _combine_body,
      grid=(grid,),
      in_specs=[
          pl.BlockSpec((_BLK, 2 * _D), lambda i: (i, 0)),
          pl.BlockSpec((_BLK, 2 * _D), lambda i: (i, 0)),
          pl.BlockSpec((_BLK, 2 * _D), lambda i: (i, 0)),
          pl.BlockSpec((_BLK, _D), lambda i: (i, 0)),
          pl.BlockSpec((_BLK, 8), lambda i: (i, 0)),
          pl.BlockSpec((2 * _D, _D), lambda i: (0, 0)),
          pl.BlockSpec((2 * _D, _D), lambda i: (0, 0)),
          pl.BlockSpec((2 * _D, _D), lambda i: (0, 0)),
          pl.BlockSpec((_D, _D), lambda i: (0, 0)),
          pl.BlockSpec((2, _D), lambda i: (0, 0)),
          pl.BlockSpec((1, _D), lambda i: (0, 0)),
      ],
      out_specs=pl.BlockSpec((_BLK, _D), lambda i: (i, 0)),
      out_shape=jax.ShapeDtypeStruct((_N, _D), jnp.float32),
  )(arel2, aent2, atim2, ent_emb, aux, wr, ws, wt, wself, bio, bs)


def _lin_body(x_ref, w_ref, b_ref, o_ref):
  o_ref[...] = jnp.dot(x_ref[...], w_ref[...], precision=lax.Precision.HIGHEST,
                       preferred_element_type=jnp.float32) + b_ref[...]


def _tc_lin(x, wt, b):
  return pl.pallas_call(
      _lin_body,
      out_shape=jax.ShapeDtypeStruct((x.shape[0], _D), jnp.float32),
  )(x, wt, b)


def kernel(ent_emb, rel_emb, time_emb, edge_index, b_rel, e_time, inv, Wi_w,
           Wi_b, Wo_w, Wo_b, Ws_w, Ws_b, Wr_w, Wr_b, Wt_w, Wt_b):
  src = edge_index[0]
  dst = edge_index[1]
  comb = jnp.concatenate([rel_emb, ent_emb], axis=0)
  acc_re = _sc_agg_ab(dst, inv, b_rel, src, comb)
  atim, cnt = _sc_agg_c(dst, inv, e_time, time_emb)
  arel2 = acc_re[:, :_OUT_ROWS].reshape(_N, 2 * _D)
  aent2 = acc_re[:, _ROWS:_ROWS + _OUT_ROWS].reshape(_N, 2 * _D)
  atim2 = atim[:, :_OUT_ROWS].reshape(_N, 2 * _D)
  cnt2 = cnt[:, :_OUT_ROWS, 0].reshape(_N, 2)
  ci = cnt2[:, 0:1]
  co = cnt2[:, 1:2]
  invd = 1.0 / jnp.maximum(ci + co, 1.0)
  aux = jnp.concatenate([ci, co, invd, jnp.zeros((_N, 5), jnp.float32)],
                        axis=1)
  wr = jnp.concatenate([Wi_w[:, 0:_D].T, Wo_w[:, 0:_D].T], axis=0)
  ws = jnp.concatenate([Wi_w[:, _D:2 * _D].T, Wo_w[:, _D:2 * _D].T], axis=0)
  wt = jnp.concatenate([Wi_w[:, 2 * _D:].T, Wo_w[:, 2 * _D:].T], axis=0)
  bio = jnp.stack([Wi_b, Wo_b])
  ent_new = _tc_combine(arel2, aent2, atim2, ent_emb, aux, wr, ws, wt,
                        Ws_w.T, bio, Ws_b[None])
  rel_new = _tc_lin(rel_emb, Wr_w.T, Wr_b[None])
  time_new = _tc_lin(time_emb, Wt_w.T, Wt_b[None])
  return ent_new, rel_new, time_new


# speedup vs baseline: 1.9819x; 1.9512x over previous
import jax
import jax.numpy as jnp
from jax import lax
from jax.experimental import pallas as pl
from jax.experimental.pallas import tpu as pltpu
from jax.experimental.pallas import tpu_sc as plsc

_N = 10000
_E = 320000
_D = 128
_ROWS = 10240
_G = 80
_SB = 4000
_NSUB = 16
_EPS = _E // _NSUB
_NSB = _EPS // _SB
_TPB = _SB // _G
_ZR = 16
_RPS = _ROWS // _NSUB


def _make_sc_sum():
  def body(dst_h, inv_h, tabi_h, tab_h, out_h,
           st_dst, st_inv, st_tab,
           b_acc0, b_idx0, b_acc1, b_idx1,
           g0, g1, zbuf, acc_sh, sg0, sg1):
    c = lax.axis_index("c")
    s = lax.axis_index("s")
    i16 = lax.iota(jnp.int32, 16)
    zeros16 = jnp.zeros((16,), jnp.float32)
    dummy_acc = _N + i16 * 7

    for rr_ in range(_ZR):
      for cc in range(_D // 16):
        zbuf[rr_, pl.ds(cc * 16, 16)] = zeros16

    z0 = s * _RPS
    for k in range(_RPS // _ZR):
      pltpu.sync_copy(zbuf, acc_sh.at[pl.ds(z0 + k * _ZR, _ZR)])
    plsc.subcore_barrier()

    for sb in range(_NSB):
      eb = s * _EPS + sb * _SB
      pltpu.sync_copy(dst_h.at[pl.ds(eb, _SB)], st_dst)
      pltpu.sync_copy(inv_h.at[pl.ds(eb, _SB)], st_inv)
      pltpu.sync_copy(tabi_h.at[pl.ds(eb, _SB)], st_tab)

      def _build(t, ba, bi):
        for v in range(_G // 16):
          off = t * _G + v * 16
          dv = st_dst[pl.ds(off, 16)]
          iv = st_inv[pl.ds(off, 16)]
          tv = st_tab[pl.ds(off, 16)]
          m = iv == c
          ba[pl.ds(v * 16, 16)] = jnp.where(m, dv, dummy_acc)
          bi[pl.ds(v * 16, 16)] = jnp.where(m, tv, i16)

      def _tile2(j, carry):
        t0 = j * 2
        _build(t0, b_acc0, b_idx0)
        cg0 = pltpu.async_copy(tab_h.at[b_idx0], g0, sg0)
        _build(t0 + 1, b_acc1, b_idx1)
        cg1 = pltpu.async_copy(tab_h.at[b_idx1], g1, sg1)
        cg0.wait()
        pltpu.sync_copy(g0, acc_sh.at[b_acc0], add=True)
        cg1.wait()
        pltpu.sync_copy(g1, acc_sh.at[b_acc1], add=True)
        return carry

      lax.fori_loop(0, _TPB // 2, _tile2, jnp.int32(0))

    plsc.subcore_barrier()
    rr = pl.ds(s * _RPS, _RPS)
    pltpu.sync_copy(acc_sh.at[rr], out_h.at[c, rr])
    plsc.subcore_barrier()

  return pl.kernel(
      body,
      out_type=jax.ShapeDtypeStruct((2, _ROWS, _D), jnp.float32),
      mesh=plsc.VectorSubcoreMesh(core_axis_name="c", subcore_axis_name="s"),
      scratch_types=[
          pltpu.VMEM((_SB,), jnp.int32),
          pltpu.VMEM((_SB,), jnp.int32),
          pltpu.VMEM((_SB,), jnp.int32),
          pltpu.VMEM((_G,), jnp.int32),
          pltpu.VMEM((_G,), jnp.int32),
          pltpu.VMEM((_G,), jnp.int32),
          pltpu.VMEM((_G,), jnp.int32),
          pltpu.VMEM((_G, _D), jnp.float32),
          pltpu.VMEM((_G, _D), jnp.float32),
          pltpu.VMEM((_ZR, _D), jnp.float32),
          pltpu.VMEM_SHARED((_ROWS, _D), jnp.float32),
          pltpu.SemaphoreType.DMA,
          pltpu.SemaphoreType.DMA,
      ],
  )


_sc_sum = _make_sc_sum()


def _sc_cnt_body(dst_h, inv_h, cnt_h,
                 st_dst, st_inv,
                 b_acc0, b_acc1, ones_b, zbuf, cnt_sh):
  c = lax.axis_index("c")
  s = lax.axis_index("s")
  i16 = lax.iota(jnp.int32, 16)
  ones16 = jnp.ones((16,), jnp.float32)
  zeros16 = jnp.zeros((16,), jnp.float32)
  dummy_acc = _N + i16 * 7

  for rr_ in range(_ZR):
    for cc in range(_D // 16):
      zbuf[rr_, pl.ds(cc * 16, 16)] = zeros16
  for rr_ in range(_G):
    for cc in range(_D // 16):
      ones_b[rr_, pl.ds(cc * 16, 16)] = ones16

  z0 = s * _RPS
  for k in range(_RPS // _ZR):
    pltpu.sync_copy(zbuf, cnt_sh.at[pl.ds(z0 + k * _ZR, _ZR)])
  plsc.subcore_barrier()

  for sb in range(_NSB):
    eb = s * _EPS + sb * _SB
    pltpu.sync_copy(dst_h.at[pl.ds(eb, _SB)], st_dst)
    pltpu.sync_copy(inv_h.at[pl.ds(eb, _SB)], st_inv)

    def _build(t, ba):
      for v in range(_G // 16):
        off = t * _G + v * 16
        dv = st_dst[pl.ds(off, 16)]
        iv = st_inv[pl.ds(off, 16)]
        m = iv == c
        ba[pl.ds(v * 16, 16)] = jnp.where(m, dv, dummy_acc)

    def _tile2(j, carry):
      t0 = j * 2
      _build(t0, b_acc0)
      pltpu.sync_copy(ones_b, cnt_sh.at[b_acc0], add=True)
      _build(t0 + 1, b_acc1)
      pltpu.sync_copy(ones_b, cnt_sh.at[b_acc1], add=True)
      return carry

    lax.fori_loop(0, _TPB // 2, _tile2, jnp.int32(0))

  plsc.subcore_barrier()
  rr = pl.ds(s * _RPS, _RPS)
  pltpu.sync_copy(cnt_sh.at[rr], cnt_h.at[c, rr])
  plsc.subcore_barrier()


_sc_cnt = pl.kernel(
    _sc_cnt_body,
    out_type=jax.ShapeDtypeStruct((2, _ROWS, _D), jnp.float32),
    mesh=plsc.VectorSubcoreMesh(core_axis_name="c", subcore_axis_name="s"),
    scratch_types=[
        pltpu.VMEM((_SB,), jnp.int32),
        pltpu.VMEM((_SB,), jnp.int32),
        pltpu.VMEM((_G,), jnp.int32),
        pltpu.VMEM((_G,), jnp.int32),
        pltpu.VMEM((_G, _D), jnp.float32),
        pltpu.VMEM((_ZR, _D), jnp.float32),
        pltpu.VMEM_SHARED((_ROWS, _D), jnp.float32),
    ],
)


_BLK = 1000


def _combine_body(arel_ref, aent_ref, atim_ref, ent_ref, aux_ref, wr_ref,
                  ws_ref, wt_ref, wself_ref, bio_ref, bs_ref, o_ref):
  hi = lax.Precision.HIGHEST
  acc = jnp.dot(arel_ref[...], wr_ref[...], precision=hi,
                preferred_element_type=jnp.float32)
  acc = acc + jnp.dot(aent_ref[...], ws_ref[...], precision=hi,
                      preferred_element_type=jnp.float32)
  acc = acc + jnp.dot(atim_ref[...], wt_ref[...], precision=hi,
                      preferred_element_type=jnp.float32)
  aux = aux_ref[...]
  bio = bio_ref[...]
  acc = acc + aux[:, 0:1] * bio[0:1, :] + aux[:, 1:2] * bio[1:2, :]
  acc = acc * aux[:, 2:3]
  o_ref[...] = acc + jnp.dot(ent_ref[...], wself_ref[...], precision=hi,
                             preferred_element_type=jnp.float32) + bs_ref[...]


def _tc_combine(arel2, aent2, atim2, ent_emb, aux, wr, ws, wt, wself, bio,
                bs):
  grid = _N // _BLK
  return pl.pallas_call(
      _combine_body,
      grid=(grid,),
      in_specs=[
          pl.BlockSpec((_BLK, 2 * _D), lambda i: (i, 0)),
          pl.BlockSpec((_BLK, 2 * _D), lambda i: (i, 0)),
          pl.BlockSpec((_BLK, 2 * _D), lambda i: (i, 0)),
          pl.BlockSpec((_BLK, _D), lambda i: (i, 0)),
          pl.BlockSpec((_BLK, 8), lambda i: (i, 0)),
          pl.BlockSpec((2 * _D, _D), lambda i: (0, 0)),
          pl.BlockSpec((2 * _D, _D), lambda i: (0, 0)),
          pl.BlockSpec((2 * _D, _D), lambda i: (0, 0)),
          pl.BlockSpec((_D, _D), lambda i: (0, 0)),
          pl.BlockSpec((2, _D), lambda i: (0, 0)),
          pl.BlockSpec((1, _D), lambda i: (0, 0)),
      ],
      out_specs=pl.BlockSpec((_BLK, _D), lambda i: (i, 0)),
      out_shape=jax.ShapeDtypeStruct((_N, _D), jnp.float32),
  )(arel2, aent2, atim2, ent_emb, aux, wr, ws, wt, wself, bio, bs)


def _lin_body(x_ref, w_ref, b_ref, o_ref):
  o_ref[...] = jnp.dot(x_ref[...], w_ref[...], precision=lax.Precision.HIGHEST,
                       preferred_element_type=jnp.float32) + b_ref[...]


def _tc_lin(x, wt, b):
  return pl.pallas_call(
      _lin_body,
      out_shape=jax.ShapeDtypeStruct((x.shape[0], _D), jnp.float32),
  )(x, wt, b)


def kernel(ent_emb, rel_emb, time_emb, edge_index, b_rel, e_time, inv, Wi_w,
           Wi_b, Wo_w, Wo_b, Ws_w, Ws_b, Wr_w, Wr_b, Wt_w, Wt_b):
  src = edge_index[0]
  dst = edge_index[1]
  arel = _sc_sum(dst, inv, b_rel, rel_emb)
  aent = _sc_sum(dst, inv, src, ent_emb)
  atim = _sc_sum(dst, inv, e_time, time_emb)
  cnt = _sc_cnt(dst, inv)
  arel2 = jnp.concatenate([arel[0, :_N], arel[1, :_N]], axis=1)
  aent2 = jnp.concatenate([aent[0, :_N], aent[1, :_N]], axis=1)
  atim2 = jnp.concatenate([atim[0, :_N], atim[1, :_N]], axis=1)
  ci = cnt[0, :_N, 0:1]
  co = cnt[1, :_N, 0:1]
  invd = 1.0 / jnp.maximum(ci + co, 1.0)
  aux = jnp.concatenate([ci, co, invd, jnp.zeros((_N, 5), jnp.float32)],
                        axis=1)
  wr = jnp.concatenate([Wi_w[:, 0:_D].T, Wo_w[:, 0:_D].T], axis=0)
  ws = jnp.concatenate([Wi_w[:, _D:2 * _D].T, Wo_w[:, _D:2 * _D].T], axis=0)
  wt = jnp.concatenate([Wi_w[:, 2 * _D:].T, Wo_w[:, 2 * _D:].T], axis=0)
  bio = jnp.stack([Wi_b, Wo_b])
  ent_new = _tc_combine(arel2, aent2, atim2, ent_emb, aux, wr, ws, wt,
                        Ws_w.T, bio, Ws_b[None])
  rel_new = _tc_lin(rel_emb, Wr_w.T, Wr_b[None])
  time_new = _tc_lin(time_emb, Wt_w.T, Wt_b[None])
  return ent_new, rel_new, time_new


# speedup vs baseline: 1.9836x; 1.0009x over previous
import jax
import jax.numpy as jnp
from jax import lax
from jax.experimental import pallas as pl
from jax.experimental.pallas import tpu as pltpu
from jax.experimental.pallas import tpu_sc as plsc

_N = 10000
_E = 320000
_D = 128
_ROWS = 10240
_G = 80
_SB = 4000
_NSUB = 16
_EPS = _E // _NSUB
_NSB = _EPS // _SB
_TPB = _SB // _G
_ZR = 16
_RPS = _ROWS // _NSUB


def _make_sc_sum():
  def body(dst_h, inv_h, tabi_h, tab_h, out_h,
           st_dst, st_inv, st_tab,
           b_acc0, b_idx0, b_acc1, b_idx1,
           g0, g1, zbuf, acc_sh, sg0, sg1, ss0, ss1):
    c = lax.axis_index("c")
    s = lax.axis_index("s")
    i16 = lax.iota(jnp.int32, 16)
    zeros16 = jnp.zeros((16,), jnp.float32)
    dummy_acc = _N + i16 * 7

    for rr_ in range(_ZR):
      for cc in range(_D // 16):
        zbuf[rr_, pl.ds(cc * 16, 16)] = zeros16

    z0 = s * _RPS
    for k in range(_RPS // _ZR):
      pltpu.sync_copy(zbuf, acc_sh.at[pl.ds(z0 + k * _ZR, _ZR)])
    plsc.subcore_barrier()

    for sb in range(_NSB):
      eb = s * _EPS + sb * _SB
      pltpu.sync_copy(dst_h.at[pl.ds(eb, _SB)], st_dst)
      pltpu.sync_copy(inv_h.at[pl.ds(eb, _SB)], st_inv)
      pltpu.sync_copy(tabi_h.at[pl.ds(eb, _SB)], st_tab)

      def _build(t, ba, bi):
        for v in range(_G // 16):
          off = t * _G + v * 16
          dv = st_dst[pl.ds(off, 16)]
          iv = st_inv[pl.ds(off, 16)]
          tv = st_tab[pl.ds(off, 16)]
          m = iv == c
          ba[pl.ds(v * 16, 16)] = jnp.where(m, dv, dummy_acc)
          bi[pl.ds(v * 16, 16)] = jnp.where(m, tv, i16)

      def _tile2(j, carry):
        t0 = j * 2
        _build(t0, b_acc0, b_idx0)
        cg0 = pltpu.async_copy(tab_h.at[b_idx0], g0, sg0)
        _build(t0 + 1, b_acc1, b_idx1)
        cg1 = pltpu.async_copy(tab_h.at[b_idx1], g1, sg1)
        cg0.wait()
        cs0 = pltpu.async_copy(g0, acc_sh.at[b_acc0], ss0, add=True)
        cg1.wait()
        cs1 = pltpu.async_copy(g1, acc_sh.at[b_acc1], ss1, add=True)
        cs0.wait()
        cs1.wait()
        return carry

      lax.fori_loop(0, _TPB // 2, _tile2, jnp.int32(0))

    plsc.subcore_barrier()
    rr = pl.ds(s * _RPS, _RPS)
    pltpu.sync_copy(acc_sh.at[rr], out_h.at[c, rr])
    plsc.subcore_barrier()

  return pl.kernel(
      body,
      out_type=jax.ShapeDtypeStruct((2, _ROWS, _D), jnp.float32),
      mesh=plsc.VectorSubcoreMesh(core_axis_name="c", subcore_axis_name="s"),
      scratch_types=[
          pltpu.VMEM((_SB,), jnp.int32),
          pltpu.VMEM((_SB,), jnp.int32),
          pltpu.VMEM((_SB,), jnp.int32),
          pltpu.VMEM((_G,), jnp.int32),
          pltpu.VMEM((_G,), jnp.int32),
          pltpu.VMEM((_G,), jnp.int32),
          pltpu.VMEM((_G,), jnp.int32),
          pltpu.VMEM((_G, _D), jnp.float32),
          pltpu.VMEM((_G, _D), jnp.float32),
          pltpu.VMEM((_ZR, _D), jnp.float32),
          pltpu.VMEM_SHARED((_ROWS, _D), jnp.float32),
          pltpu.SemaphoreType.DMA,
          pltpu.SemaphoreType.DMA,
          pltpu.SemaphoreType.DMA,
          pltpu.SemaphoreType.DMA,
      ],
  )


_sc_sum = _make_sc_sum()


def _sc_cnt_body(dst_h, inv_h, cnt_h,
                 st_dst, st_inv,
                 b_acc0, b_acc1, ones_b, zbuf, cnt_sh, ss0, ss1):
  c = lax.axis_index("c")
  s = lax.axis_index("s")
  i16 = lax.iota(jnp.int32, 16)
  ones16 = jnp.ones((16,), jnp.float32)
  zeros16 = jnp.zeros((16,), jnp.float32)
  dummy_acc = _N + i16 * 7

  for rr_ in range(_ZR):
    for cc in range(_D // 16):
      zbuf[rr_, pl.ds(cc * 16, 16)] = zeros16
  for rr_ in range(_G):
    for cc in range(_D // 16):
      ones_b[rr_, pl.ds(cc * 16, 16)] = ones16

  z0 = s * _RPS
  for k in range(_RPS // _ZR):
    pltpu.sync_copy(zbuf, cnt_sh.at[pl.ds(z0 + k * _ZR, _ZR)])
  plsc.subcore_barrier()

  for sb in range(_NSB):
    eb = s * _EPS + sb * _SB
    pltpu.sync_copy(dst_h.at[pl.ds(eb, _SB)], st_dst)
    pltpu.sync_copy(inv_h.at[pl.ds(eb, _SB)], st_inv)

    def _build(t, ba):
      for v in range(_G // 16):
        off = t * _G + v * 16
        dv = st_dst[pl.ds(off, 16)]
        iv = st_inv[pl.ds(off, 16)]
        m = iv == c
        ba[pl.ds(v * 16, 16)] = jnp.where(m, dv, dummy_acc)

    def _tile2(j, carry):
      t0 = j * 2
      _build(t0, b_acc0)
      cs0 = pltpu.async_copy(ones_b, cnt_sh.at[b_acc0], ss0, add=True)
      _build(t0 + 1, b_acc1)
      cs1 = pltpu.async_copy(ones_b, cnt_sh.at[b_acc1], ss1, add=True)
      cs0.wait()
      cs1.wait()
      return carry

    lax.fori_loop(0, _TPB // 2, _tile2, jnp.int32(0))

  plsc.subcore_barrier()
  rr = pl.ds(s * _RPS, _RPS)
  pltpu.sync_copy(cnt_sh.at[rr], cnt_h.at[c, rr])
  plsc.subcore_barrier()


_sc_cnt = pl.kernel(
    _sc_cnt_body,
    out_type=jax.ShapeDtypeStruct((2, _ROWS, _D), jnp.float32),
    mesh=plsc.VectorSubcoreMesh(core_axis_name="c", subcore_axis_name="s"),
    scratch_types=[
        pltpu.VMEM((_SB,), jnp.int32),
        pltpu.VMEM((_SB,), jnp.int32),
        pltpu.VMEM((_G,), jnp.int32),
        pltpu.VMEM((_G,), jnp.int32),
        pltpu.VMEM((_G, _D), jnp.float32),
        pltpu.VMEM((_ZR, _D), jnp.float32),
        pltpu.VMEM_SHARED((_ROWS, _D), jnp.float32),
        pltpu.SemaphoreType.DMA,
        pltpu.SemaphoreType.DMA,
    ],
)


_BLK = 1000


def _combine_body(arel_ref, aent_ref, atim_ref, ent_ref, aux_ref, wr_ref,
                  ws_ref, wt_ref, wself_ref, bio_ref, bs_ref, o_ref):
  hi = lax.Precision.HIGHEST
  acc = jnp.dot(arel_ref[...], wr_ref[...], precision=hi,
                preferred_element_type=jnp.float32)
  acc = acc + jnp.dot(aent_ref[...], ws_ref[...], precision=hi,
                      preferred_element_type=jnp.float32)
  acc = acc + jnp.dot(atim_ref[...], wt_ref[...], precision=hi,
                      preferred_element_type=jnp.float32)
  aux = aux_ref[...]
  bio = bio_ref[...]
  acc = acc + aux[:, 0:1] * bio[0:1, :] + aux[:, 1:2] * bio[1:2, :]
  acc = acc * aux[:, 2:3]
  o_ref[...] = acc + jnp.dot(ent_ref[...], wself_ref[...], precision=hi,
                             preferred_element_type=jnp.float32) + bs_ref[...]


def _tc_combine(arel2, aent2, atim2, ent_emb, aux, wr, ws, wt, wself, bio,
                bs):
  grid = _N // _BLK
  return pl.pallas_call(
      _combine_body,
      grid=(grid,),
      in_specs=[
          pl.BlockSpec((_BLK, 2 * _D), lambda i: (i, 0)),
          pl.BlockSpec((_BLK, 2 * _D), lambda i: (i, 0)),
          pl.BlockSpec((_BLK, 2 * _D), lambda i: (i, 0)),
          pl.BlockSpec((_BLK, _D), lambda i: (i, 0)),
          pl.BlockSpec((_BLK, 8), lambda i: (i, 0)),
          pl.BlockSpec((2 * _D, _D), lambda i: (0, 0)),
          pl.BlockSpec((2 * _D, _D), lambda i: (0, 0)),
          pl.BlockSpec((2 * _D, _D), lambda i: (0, 0)),
          pl.BlockSpec((_D, _D), lambda i: (0, 0)),
          pl.BlockSpec((2, _D), lambda i: (0, 0)),
          pl.BlockSpec((1, _D), lambda i: (0, 0)),
      ],
      out_specs=pl.BlockSpec((_BLK, _D), lambda i: (i, 0)),
      out_shape=jax.ShapeDtypeStruct((_N, _D), jnp.float32),
  )(arel2, aent2, atim2, ent_emb, aux, wr, ws, wt, wself, bio, bs)


def _lin_body(x_ref, w_ref, b_ref, o_ref):
  o_ref[...] = jnp.dot(x_ref[...], w_ref[...], precision=lax.Precision.HIGHEST,
                       preferred_element_type=jnp.float32) + b_ref[...]


def _tc_lin(x, wt, b):
  return pl.pallas_call(
      _lin_body,
      out_shape=jax.ShapeDtypeStruct((x.shape[0], _D), jnp.float32),
  )(x, wt, b)


def kernel(ent_emb, rel_emb, time_emb, edge_index, b_rel, e_time, inv, Wi_w,
           Wi_b, Wo_w, Wo_b, Ws_w, Ws_b, Wr_w, Wr_b, Wt_w, Wt_b):
  src = edge_index[0]
  dst = edge_index[1]
  arel = _sc_sum(dst, inv, b_rel, rel_emb)
  aent = _sc_sum(dst, inv, src, ent_emb)
  atim = _sc_sum(dst, inv, e_time, time_emb)
  cnt = _sc_cnt(dst, inv)
  arel2 = jnp.concatenate([arel[0, :_N], arel[1, :_N]], axis=1)
  aent2 = jnp.concatenate([aent[0, :_N], aent[1, :_N]], axis=1)
  atim2 = jnp.concatenate([atim[0, :_N], atim[1, :_N]], axis=1)
  ci = cnt[0, :_N, 0:1]
  co = cnt[1, :_N, 0:1]
  invd = 1.0 / jnp.maximum(ci + co, 1.0)
  aux = jnp.concatenate([ci, co, invd, jnp.zeros((_N, 5), jnp.float32)],
                        axis=1)
  wr = jnp.concatenate([Wi_w[:, 0:_D].T, Wo_w[:, 0:_D].T], axis=0)
  ws = jnp.concatenate([Wi_w[:, _D:2 * _D].T, Wo_w[:, _D:2 * _D].T], axis=0)
  wt = jnp.concatenate([Wi_w[:, 2 * _D:].T, Wo_w[:, 2 * _D:].T], axis=0)
  bio = jnp.stack([Wi_b, Wo_b])
  ent_new = _tc_combine(arel2, aent2, atim2, ent_emb, aux, wr, ws, wt,
                        Ws_w.T, bio, Ws_b[None])
  rel_new = _tc_lin(rel_emb, Wr_w.T, Wr_b[None])
  time_new = _tc_lin(time_emb, Wt_w.T, Wt_b[None])
  return ent_new, rel_new, time_new


# speedup vs baseline: 6.1859x; 3.1185x over previous
import jax
import jax.numpy as jnp
from jax import lax
from jax.experimental import pallas as pl
from jax.experimental.pallas import tpu as pltpu
from jax.experimental.pallas import tpu_sc as plsc

_N = 10000
_E = 320000
_D = 128
_ROWS = 10240
_G = 80
_SB = 4000
_NSUB = 16
_EPS = _E // _NSUB
_NSB = _EPS // _SB
_TPB = _SB // _G
_ZR = 16
_RPS = _ROWS // _NSUB


def _make_sc_sum(nrows):
  def body(dst_h, inv_h, tabi_h, tab_h, out_h,
           st_dst, st_inv, st_tab,
           b_acc0, b_idx0, b_acc1, b_idx1,
           g0, g1, zbuf, acc_sh, sg0, sg1, ss0, ss1):
    c = lax.axis_index("c")
    s = lax.axis_index("s")
    i16 = lax.iota(jnp.int32, 16)
    zeros16 = jnp.zeros((16,), jnp.float32)
    dummy_acc = _N + i16 * 7

    for rr_ in range(_ZR):
      for cc in range(_D // 16):
        zbuf[rr_, pl.ds(cc * 16, 16)] = zeros16

    z0 = s * _RPS
    for k in range(_RPS // _ZR):
      pltpu.sync_copy(zbuf, acc_sh.at[pl.ds(z0 + k * _ZR, _ZR)])
    plsc.subcore_barrier()

    for sb in range(_NSB):
      eb = s * _EPS + sb * _SB
      pltpu.sync_copy(dst_h.at[pl.ds(eb, _SB)], st_dst)
      pltpu.sync_copy(inv_h.at[pl.ds(eb, _SB)], st_inv)
      pltpu.sync_copy(tabi_h.at[pl.ds(eb, _SB)], st_tab)

      def _build(t, ba, bi):
        for v in range(_G // 16):
          off = t * _G + v * 16
          dv = st_dst[pl.ds(off, 16)]
          iv = st_inv[pl.ds(off, 16)]
          tv = st_tab[pl.ds(off, 16)]
          m = iv == c
          didx = lax.rem(i16 * 7 + t * 89 + v * 23 + s * 41, nrows)
          ba[pl.ds(v * 16, 16)] = jnp.where(m, dv, dummy_acc)
          bi[pl.ds(v * 16, 16)] = jnp.where(m, tv, didx)

      def _tile2(j, carry):
        t0 = j * 2
        _build(t0, b_acc0, b_idx0)
        cg0 = pltpu.async_copy(tab_h.at[b_idx0], g0, sg0)
        _build(t0 + 1, b_acc1, b_idx1)
        cg1 = pltpu.async_copy(tab_h.at[b_idx1], g1, sg1)
        cg0.wait()
        cs0 = pltpu.async_copy(g0, acc_sh.at[b_acc0], ss0, add=True)
        cg1.wait()
        cs1 = pltpu.async_copy(g1, acc_sh.at[b_acc1], ss1, add=True)
        cs0.wait()
        cs1.wait()
        return carry

      lax.fori_loop(0, _TPB // 2, _tile2, jnp.int32(0))

    plsc.subcore_barrier()
    rr = pl.ds(s * _RPS, _RPS)
    pltpu.sync_copy(acc_sh.at[rr], out_h.at[c, rr])
    plsc.subcore_barrier()

  return pl.kernel(
      body,
      out_type=jax.ShapeDtypeStruct((2, _ROWS, _D), jnp.float32),
      mesh=plsc.VectorSubcoreMesh(core_axis_name="c", subcore_axis_name="s"),
      scratch_types=[
          pltpu.VMEM((_SB,), jnp.int32),
          pltpu.VMEM((_SB,), jnp.int32),
          pltpu.VMEM((_SB,), jnp.int32),
          pltpu.VMEM((_G,), jnp.int32),
          pltpu.VMEM((_G,), jnp.int32),
          pltpu.VMEM((_G,), jnp.int32),
          pltpu.VMEM((_G,), jnp.int32),
          pltpu.VMEM((_G, _D), jnp.float32),
          pltpu.VMEM((_G, _D), jnp.float32),
          pltpu.VMEM((_ZR, _D), jnp.float32),
          pltpu.VMEM_SHARED((_ROWS, _D), jnp.float32),
          pltpu.SemaphoreType.DMA,
          pltpu.SemaphoreType.DMA,
          pltpu.SemaphoreType.DMA,
          pltpu.SemaphoreType.DMA,
      ],
  )


_sc_sum_rel = _make_sc_sum(500)
_sc_sum_ent = _make_sc_sum(_N)
_sc_sum_tim = _make_sc_sum(366)


def _sc_cnt_body(dst_h, inv_h, cnt_h,
                 st_dst, st_inv,
                 b_acc0, b_acc1, ones_b, zbuf, cnt_sh, ss0, ss1):
  c = lax.axis_index("c")
  s = lax.axis_index("s")
  i16 = lax.iota(jnp.int32, 16)
  ones16 = jnp.ones((16,), jnp.float32)
  zeros16 = jnp.zeros((16,), jnp.float32)
  dummy_acc = _N + i16 * 7

  for rr_ in range(_ZR):
    for cc in range(_D // 16):
      zbuf[rr_, pl.ds(cc * 16, 16)] = zeros16
  for rr_ in range(_G):
    for cc in range(_D // 16):
      ones_b[rr_, pl.ds(cc * 16, 16)] = ones16

  z0 = s * _RPS
  for k in range(_RPS // _ZR):
    pltpu.sync_copy(zbuf, cnt_sh.at[pl.ds(z0 + k * _ZR, _ZR)])
  plsc.subcore_barrier()

  for sb in range(_NSB):
    eb = s * _EPS + sb * _SB
    pltpu.sync_copy(dst_h.at[pl.ds(eb, _SB)], st_dst)
    pltpu.sync_copy(inv_h.at[pl.ds(eb, _SB)], st_inv)

    def _build(t, ba):
      for v in range(_G // 16):
        off = t * _G + v * 16
        dv = st_dst[pl.ds(off, 16)]
        iv = st_inv[pl.ds(off, 16)]
        m = iv == c
        ba[pl.ds(v * 16, 16)] = jnp.where(m, dv, dummy_acc)

    def _tile2(j, carry):
      t0 = j * 2
      _build(t0, b_acc0)
      cs0 = pltpu.async_copy(ones_b, cnt_sh.at[b_acc0], ss0, add=True)
      _build(t0 + 1, b_acc1)
      cs1 = pltpu.async_copy(ones_b, cnt_sh.at[b_acc1], ss1, add=True)
      cs0.wait()
      cs1.wait()
      return carry

    lax.fori_loop(0, _TPB // 2, _tile2, jnp.int32(0))

  plsc.subcore_barrier()
  rr = pl.ds(s * _RPS, _RPS)
  pltpu.sync_copy(cnt_sh.at[rr], cnt_h.at[c, rr])
  plsc.subcore_barrier()


_sc_cnt = pl.kernel(
    _sc_cnt_body,
    out_type=jax.ShapeDtypeStruct((2, _ROWS, _D), jnp.float32),
    mesh=plsc.VectorSubcoreMesh(core_axis_name="c", subcore_axis_name="s"),
    scratch_types=[
        pltpu.VMEM((_SB,), jnp.int32),
        pltpu.VMEM((_SB,), jnp.int32),
        pltpu.VMEM((_G,), jnp.int32),
        pltpu.VMEM((_G,), jnp.int32),
        pltpu.VMEM((_G, _D), jnp.float32),
        pltpu.VMEM((_ZR, _D), jnp.float32),
        pltpu.VMEM_SHARED((_ROWS, _D), jnp.float32),
        pltpu.SemaphoreType.DMA,
        pltpu.SemaphoreType.DMA,
    ],
)


_BLK = 1000


def _combine_body(arel_ref, aent_ref, atim_ref, ent_ref, aux_ref, wr_ref,
                  ws_ref, wt_ref, wself_ref, bio_ref, bs_ref, o_ref):
  hi = lax.Precision.HIGHEST
  acc = jnp.dot(arel_ref[...], wr_ref[...], precision=hi,
                preferred_element_type=jnp.float32)
  acc = acc + jnp.dot(aent_ref[...], ws_ref[...], precision=hi,
                      preferred_element_type=jnp.float32)
  acc = acc + jnp.dot(atim_ref[...], wt_ref[...], precision=hi,
                      preferred_element_type=jnp.float32)
  aux = aux_ref[...]
  bio = bio_ref[...]
  acc = acc + aux[:, 0:1] * bio[0:1, :] + aux[:, 1:2] * bio[1:2, :]
  acc = acc * aux[:, 2:3]
  o_ref[...] = acc + jnp.dot(ent_ref[...], wself_ref[...], precision=hi,
                             preferred_element_type=jnp.float32) + bs_ref[...]


def _tc_combine(arel2, aent2, atim2, ent_emb, aux, wr, ws, wt, wself, bio,
                bs):
  grid = _N // _BLK
  return pl.pallas_call(
      _combine_body,
      grid=(grid,),
      in_specs=[
          pl.BlockSpec((_BLK, 2 * _D), lambda i: (i, 0)),
          pl.BlockSpec((_BLK, 2 * _D), lambda i: (i, 0)),
          pl.BlockSpec((_BLK, 2 * _D), lambda i: (i, 0)),
          pl.BlockSpec((_BLK, _D), lambda i: (i, 0)),
          pl.BlockSpec((_BLK, 8), lambda i: (i, 0)),
          pl.BlockSpec((2 * _D, _D), lambda i: (0, 0)),
          pl.BlockSpec((2 * _D, _D), lambda i: (0, 0)),
          pl.BlockSpec((2 * _D, _D), lambda i: (0, 0)),
          pl.BlockSpec((_D, _D), lambda i: (0, 0)),
          pl.BlockSpec((2, _D), lambda i: (0, 0)),
          pl.BlockSpec((1, _D), lambda i: (0, 0)),
      ],
      out_specs=pl.BlockSpec((_BLK, _D), lambda i: (i, 0)),
      out_shape=jax.ShapeDtypeStruct((_N, _D), jnp.float32),
  )(arel2, aent2, atim2, ent_emb, aux, wr, ws, wt, wself, bio, bs)


def _lin_body(x_ref, w_ref, b_ref, o_ref):
  o_ref[...] = jnp.dot(x_ref[...], w_ref[...], precision=lax.Precision.HIGHEST,
                       preferred_element_type=jnp.float32) + b_ref[...]


def _tc_lin(x, wt, b):
  return pl.pallas_call(
      _lin_body,
      out_shape=jax.ShapeDtypeStruct((x.shape[0], _D), jnp.float32),
  )(x, wt, b)


def kernel(ent_emb, rel_emb, time_emb, edge_index, b_rel, e_time, inv, Wi_w,
           Wi_b, Wo_w, Wo_b, Ws_w, Ws_b, Wr_w, Wr_b, Wt_w, Wt_b):
  src = edge_index[0]
  dst = edge_index[1]
  arel = _sc_sum_rel(dst, inv, b_rel, rel_emb)
  aent = _sc_sum_ent(dst, inv, src, ent_emb)
  atim = _sc_sum_tim(dst, inv, e_time, time_emb)
  cnt = _sc_cnt(dst, inv)
  arel2 = jnp.concatenate([arel[0, :_N], arel[1, :_N]], axis=1)
  aent2 = jnp.concatenate([aent[0, :_N], aent[1, :_N]], axis=1)
  atim2 = jnp.concatenate([atim[0, :_N], atim[1, :_N]], axis=1)
  ci = cnt[0, :_N, 0:1]
  co = cnt[1, :_N, 0:1]
  invd = 1.0 / jnp.maximum(ci + co, 1.0)
  aux = jnp.concatenate([ci, co, invd, jnp.zeros((_N, 5), jnp.float32)],
                        axis=1)
  wr = jnp.concatenate([Wi_w[:, 0:_D].T, Wo_w[:, 0:_D].T], axis=0)
  ws = jnp.concatenate([Wi_w[:, _D:2 * _D].T, Wo_w[:, _D:2 * _D].T], axis=0)
  wt = jnp.concatenate([Wi_w[:, 2 * _D:].T, Wo_w[:, 2 * _D:].T], axis=0)
  bio = jnp.stack([Wi_b, Wo_b])
  ent_new = _tc_combine(arel2, aent2, atim2, ent_emb, aux, wr, ws, wt,
                        Ws_w.T, bio, Ws_b[None])
  rel_new = _tc_lin(rel_emb, Wr_w.T, Wr_b[None])
  time_new = _tc_lin(time_emb, Wt_w.T, Wt_b[None])
  return ent_new, rel_new, time_new


# speedup vs baseline: 6.2743x; 1.0143x over previous
import jax
import jax.numpy as jnp
from jax import lax
from jax.experimental import pallas as pl
from jax.experimental.pallas import tpu as pltpu
from jax.experimental.pallas import tpu_sc as plsc

_N = 10000
_E = 320000
_D = 128
_ROWS = 10240
_G = 80
_SB = 4000
_NSUB = 16
_EPS = _E // _NSUB
_NSB = _EPS // _SB
_TPB = _SB // _G
_ZR = 16
_RPS = _ROWS // _NSUB


def _make_sc_sum(nrows, stage_rows=0):
  def body(dst_h, inv_h, tabi_h, tab_h, out_h,
           st_dst, st_inv, st_tab,
           b_acc0, b_idx0, b_acc1, b_idx1,
           g0, g1, zbuf, acc_sh, *rest):
    if stage_rows:
      tab_sh, sg0, sg1, ss0, ss1 = rest
    else:
      sg0, sg1, ss0, ss1 = rest
    c = lax.axis_index("c")
    s = lax.axis_index("s")
    i16 = lax.iota(jnp.int32, 16)
    zeros16 = jnp.zeros((16,), jnp.float32)
    dummy_acc = _N + i16 * 7

    for rr_ in range(_ZR):
      for cc in range(_D // 16):
        zbuf[rr_, pl.ds(cc * 16, 16)] = zeros16

    z0 = s * _RPS
    for k in range(_RPS // _ZR):
      pltpu.sync_copy(zbuf, acc_sh.at[pl.ds(z0 + k * _ZR, _ZR)])
    if stage_rows:
      win = stage_rows // _NSUB
      rw = pl.ds(s * win, win)
      pltpu.sync_copy(tab_h.at[rw], tab_sh.at[rw])
    plsc.subcore_barrier()
    gsrc = tab_sh if stage_rows else tab_h

    for sb in range(_NSB):
      eb = s * _EPS + sb * _SB
      pltpu.sync_copy(dst_h.at[pl.ds(eb, _SB)], st_dst)
      pltpu.sync_copy(inv_h.at[pl.ds(eb, _SB)], st_inv)
      pltpu.sync_copy(tabi_h.at[pl.ds(eb, _SB)], st_tab)

      def _build(t, ba, bi):
        for v in range(_G // 16):
          off = t * _G + v * 16
          dv = st_dst[pl.ds(off, 16)]
          iv = st_inv[pl.ds(off, 16)]
          tv = st_tab[pl.ds(off, 16)]
          m = iv == c
          didx = lax.rem(i16 * 7 + t * 89 + v * 23 + s * 41, nrows)
          ba[pl.ds(v * 16, 16)] = jnp.where(m, dv, dummy_acc)
          bi[pl.ds(v * 16, 16)] = jnp.where(m, tv, didx)

      def _tile2(j, carry):
        t0 = j * 2
        _build(t0, b_acc0, b_idx0)
        cg0 = pltpu.async_copy(gsrc.at[b_idx0], g0, sg0)
        _build(t0 + 1, b_acc1, b_idx1)
        cg1 = pltpu.async_copy(gsrc.at[b_idx1], g1, sg1)
        cg0.wait()
        cs0 = pltpu.async_copy(g0, acc_sh.at[b_acc0], ss0, add=True)
        cg1.wait()
        cs1 = pltpu.async_copy(g1, acc_sh.at[b_acc1], ss1, add=True)
        cs0.wait()
        cs1.wait()
        return carry

      lax.fori_loop(0, _TPB // 2, _tile2, jnp.int32(0))

    plsc.subcore_barrier()
    rr = pl.ds(s * _RPS, _RPS)
    pltpu.sync_copy(acc_sh.at[rr], out_h.at[c, rr])
    plsc.subcore_barrier()

  scratch = [
      pltpu.VMEM((_SB,), jnp.int32),
      pltpu.VMEM((_SB,), jnp.int32),
      pltpu.VMEM((_SB,), jnp.int32),
      pltpu.VMEM((_G,), jnp.int32),
      pltpu.VMEM((_G,), jnp.int32),
      pltpu.VMEM((_G,), jnp.int32),
      pltpu.VMEM((_G,), jnp.int32),
      pltpu.VMEM((_G, _D), jnp.float32),
      pltpu.VMEM((_G, _D), jnp.float32),
      pltpu.VMEM((_ZR, _D), jnp.float32),
      pltpu.VMEM_SHARED((_ROWS, _D), jnp.float32),
  ]
  if stage_rows:
    scratch.append(pltpu.VMEM_SHARED((nrows, _D), jnp.float32))
  scratch += [
      pltpu.SemaphoreType.DMA,
      pltpu.SemaphoreType.DMA,
      pltpu.SemaphoreType.DMA,
      pltpu.SemaphoreType.DMA,
  ]
  return pl.kernel(
      body,
      out_type=jax.ShapeDtypeStruct((2, _ROWS, _D), jnp.float32),
      mesh=plsc.VectorSubcoreMesh(core_axis_name="c", subcore_axis_name="s"),
      scratch_types=scratch,
  )


_sc_sum_rel = _make_sc_sum(500, stage_rows=512)
_sc_sum_ent = _make_sc_sum(_N)
_sc_sum_tim = _make_sc_sum(366, stage_rows=384)


def _sc_cnt_body(dst_h, inv_h, cnt_h,
                 st_dst, st_inv,
                 b_acc0, b_acc1, ones_b, zbuf, cnt_sh, ss0, ss1):
  c = lax.axis_index("c")
  s = lax.axis_index("s")
  i16 = lax.iota(jnp.int32, 16)
  ones16 = jnp.ones((16,), jnp.float32)
  zeros16 = jnp.zeros((16,), jnp.float32)
  dummy_acc = _N + i16 * 7

  for rr_ in range(_ZR):
    for cc in range(_D // 16):
      zbuf[rr_, pl.ds(cc * 16, 16)] = zeros16
  for rr_ in range(_G):
    for cc in range(_D // 16):
      ones_b[rr_, pl.ds(cc * 16, 16)] = ones16

  z0 = s * _RPS
  for k in range(_RPS // _ZR):
    pltpu.sync_copy(zbuf, cnt_sh.at[pl.ds(z0 + k * _ZR, _ZR)])
  plsc.subcore_barrier()

  for sb in range(_NSB):
    eb = s * _EPS + sb * _SB
    pltpu.sync_copy(dst_h.at[pl.ds(eb, _SB)], st_dst)
    pltpu.sync_copy(inv_h.at[pl.ds(eb, _SB)], st_inv)

    def _build(t, ba):
      for v in range(_G // 16):
        off = t * _G + v * 16
        dv = st_dst[pl.ds(off, 16)]
        iv = st_inv[pl.ds(off, 16)]
        m = iv == c
        ba[pl.ds(v * 16, 16)] = jnp.where(m, dv, dummy_acc)

    def _tile2(j, carry):
      t0 = j * 2
      _build(t0, b_acc0)
      cs0 = pltpu.async_copy(ones_b, cnt_sh.at[b_acc0], ss0, add=True)
      _build(t0 + 1, b_acc1)
      cs1 = pltpu.async_copy(ones_b, cnt_sh.at[b_acc1], ss1, add=True)
      cs0.wait()
      cs1.wait()
      return carry

    lax.fori_loop(0, _TPB // 2, _tile2, jnp.int32(0))

  plsc.subcore_barrier()
  rr = pl.ds(s * _RPS, _RPS)
  pltpu.sync_copy(cnt_sh.at[rr], cnt_h.at[c, rr])
  plsc.subcore_barrier()


_sc_cnt = pl.kernel(
    _sc_cnt_body,
    out_type=jax.ShapeDtypeStruct((2, _ROWS, _D), jnp.float32),
    mesh=plsc.VectorSubcoreMesh(core_axis_name="c", subcore_axis_name="s"),
    scratch_types=[
        pltpu.VMEM((_SB,), jnp.int32),
        pltpu.VMEM((_SB,), jnp.int32),
        pltpu.VMEM((_G,), jnp.int32),
        pltpu.VMEM((_G,), jnp.int32),
        pltpu.VMEM((_G, _D), jnp.float32),
        pltpu.VMEM((_ZR, _D), jnp.float32),
        pltpu.VMEM_SHARED((_ROWS, _D), jnp.float32),
        pltpu.SemaphoreType.DMA,
        pltpu.SemaphoreType.DMA,
    ],
)


_BLK = 1000


def _combine_body(arel_ref, aent_ref, atim_ref, ent_ref, aux_ref, wr_ref,
                  ws_ref, wt_ref, wself_ref, bio_ref, bs_ref, o_ref):
  hi = lax.Precision.HIGHEST
  acc = jnp.dot(arel_ref[...], wr_ref[...], precision=hi,
                preferred_element_type=jnp.float32)
  acc = acc + jnp.dot(aent_ref[...], ws_ref[...], precision=hi,
                      preferred_element_type=jnp.float32)
  acc = acc + jnp.dot(atim_ref[...], wt_ref[...], precision=hi,
                      preferred_element_type=jnp.float32)
  aux = aux_ref[...]
  bio = bio_ref[...]
  acc = acc + aux[:, 0:1] * bio[0:1, :] + aux[:, 1:2] * bio[1:2, :]
  acc = acc * aux[:, 2:3]
  o_ref[...] = acc + jnp.dot(ent_ref[...], wself_ref[...], precision=hi,
                             preferred_element_type=jnp.float32) + bs_ref[...]


def _tc_combine(arel2, aent2, atim2, ent_emb, aux, wr, ws, wt, wself, bio,
                bs):
  grid = _N // _BLK
  return pl.pallas_call(
      _combine_body,
      grid=(grid,),
      in_specs=[
          pl.BlockSpec((_BLK, 2 * _D), lambda i: (i, 0)),
          pl.BlockSpec((_BLK, 2 * _D), lambda i: (i, 0)),
          pl.BlockSpec((_BLK, 2 * _D), lambda i: (i, 0)),
          pl.BlockSpec((_BLK, _D), lambda i: (i, 0)),
          pl.BlockSpec((_BLK, 8), lambda i: (i, 0)),
          pl.BlockSpec((2 * _D, _D), lambda i: (0, 0)),
          pl.BlockSpec((2 * _D, _D), lambda i: (0, 0)),
          pl.BlockSpec((2 * _D, _D), lambda i: (0, 0)),
          pl.BlockSpec((_D, _D), lambda i: (0, 0)),
          pl.BlockSpec((2, _D), lambda i: (0, 0)),
          pl.BlockSpec((1, _D), lambda i: (0, 0)),
      ],
      out_specs=pl.BlockSpec((_BLK, _D), lambda i: (i, 0)),
      out_shape=jax.ShapeDtypeStruct((_N, _D), jnp.float32),
  )(arel2, aent2, atim2, ent_emb, aux, wr, ws, wt, wself, bio, bs)


def _lin_body(x_ref, w_ref, b_ref, o_ref):
  o_ref[...] = jnp.dot(x_ref[...], w_ref[...], precision=lax.Precision.HIGHEST,
                       preferred_element_type=jnp.float32) + b_ref[...]


def _tc_lin(x, wt, b):
  return pl.pallas_call(
      _lin_body,
      out_shape=jax.ShapeDtypeStruct((x.shape[0], _D), jnp.float32),
  )(x, wt, b)


def kernel(ent_emb, rel_emb, time_emb, edge_index, b_rel, e_time, inv, Wi_w,
           Wi_b, Wo_w, Wo_b, Ws_w, Ws_b, Wr_w, Wr_b, Wt_w, Wt_b):
  src = edge_index[0]
  dst = edge_index[1]
  arel = _sc_sum_rel(dst, inv, b_rel,
                     jnp.pad(rel_emb, ((0, 12), (0, 0))))
  aent = _sc_sum_ent(dst, inv, src, ent_emb)
  atim = _sc_sum_tim(dst, inv, e_time,
                     jnp.pad(time_emb, ((0, 18), (0, 0))))
  cnt = _sc_cnt(dst, inv)
  arel2 = jnp.concatenate([arel[0, :_N], arel[1, :_N]], axis=1)
  aent2 = jnp.concatenate([aent[0, :_N], aent[1, :_N]], axis=1)
  atim2 = jnp.concatenate([atim[0, :_N], atim[1, :_N]], axis=1)
  ci = cnt[0, :_N, 0:1]
  co = cnt[1, :_N, 0:1]
  invd = 1.0 / jnp.maximum(ci + co, 1.0)
  aux = jnp.concatenate([ci, co, invd, jnp.zeros((_N, 5), jnp.float32)],
                        axis=1)
  wr = jnp.concatenate([Wi_w[:, 0:_D].T, Wo_w[:, 0:_D].T], axis=0)
  ws = jnp.concatenate([Wi_w[:, _D:2 * _D].T, Wo_w[:, _D:2 * _D].T], axis=0)
  wt = jnp.concatenate([Wi_w[:, 2 * _D:].T, Wo_w[:, 2 * _D:].T], axis=0)
  bio = jnp.stack([Wi_b, Wo_b])
  ent_new = _tc_combine(arel2, aent2, atim2, ent_emb, aux, wr, ws, wt,
                        Ws_w.T, bio, Ws_b[None])
  rel_new = _tc_lin(rel_emb, Wr_w.T, Wr_b[None])
  time_new = _tc_lin(time_emb, Wt_w.T, Wt_b[None])
  return ent_new, rel_new, time_new
